# Initial kernel scaffold; baseline (speedup 1.0000x reference)
import jax
import jax.numpy as jnp
from jax.experimental import pallas as pl


def kernel(x, edge_index, edge_attr, W1, b1, W2, b2, W3, b3):
    raise NotImplementedError("write your pallas kernel here")


# speedup vs baseline: 5.5644x; 5.5644x over previous
import functools

import jax
import jax.numpy as jnp
from jax import lax
from jax.experimental import pallas as pl
from jax.experimental.pallas import tpu as pltpu
from jax.experimental.pallas import tpu_sc as plsc

N = 10000
E = 320000
NC = 2
NS = 16
NW = NC * NS
EPW = E // NW

CH = 125
NCH = EPW // CH
WBD = 80
NWBD = N // WBD

CHP = 128
NCHP = 80
EPWP = NCHP * CHP
PAD_DST = 20000

SPLIT = 9000
ACC_R = 9600
TR_BASE = 9024
TR_MASK = 511
WB = 64
NZ0 = 9024 // WB
NZ1 = 1024 // WB


def _mesh():
    return plsc.VectorSubcoreMesh(core_axis_name="c", subcore_axis_name="s")


def _make_degree():
    @functools.partial(
        pl.kernel,
        out_type=jax.ShapeDtypeStruct((NC, N, 16), jnp.float32),
        mesh=_mesh(),
        scratch_types=[
            pltpu.VMEM((NCH, CH), jnp.int32),
            pltpu.VMEM((CH, 16), jnp.float32),
            pltpu.VMEM_SHARED((N, 16), jnp.float32),
        ],
    )
    def deg_kernel(dst_hbm, out_hbm, idxv, buf, acc):
        c = lax.axis_index("c")
        s = lax.axis_index("s")
        wid = s * NC + c

        @pl.loop(0, CH)
        def _zero(i):
            buf[i, :] = jnp.zeros((16,), jnp.float32)

        @pl.loop(s, NWBD, step=NS)
        def _zacc(t):
            pltpu.sync_copy(buf.at[pl.ds(0, WBD)],
                            acc.at[pl.ds(t * WBD, WBD)])

        @pl.loop(0, CH)
        def _fill(i):
            buf[i, :] = jnp.ones((16,), jnp.float32)

        pltpu.sync_copy(dst_hbm.at[wid], idxv)
        plsc.subcore_barrier()

        @pl.loop(0, NCH)
        def _scatter(j):
            pltpu.sync_copy(buf, acc.at[idxv.at[j]], add=True)

        plsc.subcore_barrier()

        @pl.loop(s, NWBD, step=NS)
        def _wb(t):
            pltpu.sync_copy(acc.at[pl.ds(t * WBD, WBD)],
                            out_hbm.at[c, pl.ds(t * WBD, WBD)])

    return deg_kernel


def _make_spmm():
    @functools.partial(
        pl.kernel,
        out_type=jax.ShapeDtypeStruct((2, NC, 9024, 128), jnp.float32),
        mesh=_mesh(),
        scratch_types=[
            pltpu.VMEM((NCHP, CHP), jnp.int32),
            pltpu.VMEM((NCHP, CHP), jnp.int32),
            pltpu.VMEM((2, CHP), jnp.int32),
            pltpu.VMEM((2, CHP, 128), jnp.float32),
            pltpu.VMEM_SHARED((ACC_R, 128), jnp.float32),
            pltpu.SemaphoreType.DMA,
            pltpu.SemaphoreType.DMA,
        ],
    )
    def spmm(src_hbm, dst_hbm, y_hbm, out_hbm, srcv, dstv, idxc,
             rows, acc, sem0, sem1):
        c = lax.axis_index("c")
        s = lax.axis_index("s")
        wid = s * NC + c
        sems = (sem0, sem1)

        pltpu.sync_copy(src_hbm.at[wid], srcv)
        pltpu.sync_copy(dst_hbm.at[wid], dstv)

        @pl.loop(0, CHP)
        def _zero(i):
            for k in range(8):
                rows[0, i, pl.ds(16 * k, 16)] = jnp.zeros((16,), jnp.float32)

        def start_gather(j, b):
            pltpu.async_copy(y_hbm.at[srcv.at[j]], rows.at[b], sems[b])

        def wait_gather(j, b):
            pltpu.make_async_copy(y_hbm.at[srcv.at[j]], rows.at[b],
                                  sems[b]).wait()

        for p in range(2):
            nz = (NZ0, NZ1)[p]

            def transform(j, b, p=p):
                for k in range(8):
                    v = dstv[j, pl.ds(16 * k, 16)]
                    tr = TR_BASE + (v & TR_MASK)
                    if p == 0:
                        out = jnp.where(v < SPLIT, v, tr)
                    else:
                        ok1 = (v >= SPLIT) & (v < N)
                        out = jnp.where(ok1, v - SPLIT, tr)
                    idxc[b, pl.ds(16 * k, 16)] = out

            @pl.loop(s, nz, step=NS)
            def _zacc(t):
                pltpu.sync_copy(rows.at[0, pl.ds(0, WB)],
                                acc.at[pl.ds(t * WB, WB)])

            plsc.subcore_barrier()

            start_gather(0, 0)

            @pl.loop(0, NCHP, step=2)
            def _edges(j):
                start_gather(j + 1, 1)
                transform(j, 0)
                wait_gather(j, 0)
                pltpu.sync_copy(rows.at[0], acc.at[idxc.at[0]], add=True)

                @pl.when(j + 2 < NCHP)
                def _():
                    start_gather(j + 2, 0)

                transform(j + 1, 1)
                wait_gather(j + 1, 1)
                pltpu.sync_copy(rows.at[1], acc.at[idxc.at[1]], add=True)

            plsc.subcore_barrier()

            @pl.loop(s, nz, step=NS)
            def _wb(t):
                pltpu.sync_copy(acc.at[pl.ds(t * WB, WB)],
                                out_hbm.at[p, c, pl.ds(t * WB, WB)])

    return spmm


_deg_call = _make_degree()
_spmm = _make_spmm()


BLK = 1000


def _dinv_of(dp_ref):
    deg = dp_ref[0, :, 0:1] + dp_ref[1, :, 0:1] + 1.0
    return lax.rsqrt(deg)


def _asum_of(a_ref):
    return a_ref[0, 0] + a_ref[0, 1]


def _scale_body(dp_ref, x_ref, xs_ref):
    xs_ref[...] = x_ref[...] * _dinv_of(dp_ref)


def _layer1_body(dp_ref, x_ref, a_ref, w1_ref, b1_ref, w2_ref, ys2_ref):
    dinv = _dinv_of(dp_ref)
    g1 = dinv * (_asum_of(a_ref) + dinv * x_ref[...])
    h1 = jnp.dot(g1, w1_ref[...], preferred_element_type=jnp.float32)
    h1 = jnp.maximum(h1 + b1_ref[...], 0.0)
    ys2_ref[...] = dinv * jnp.dot(h1, w2_ref[...],
                                  preferred_element_type=jnp.float32)


def _layer2_body(dp_ref, ys2_ref, a_ref, b2_ref, w3_ref, ys3_ref):
    dinv = _dinv_of(dp_ref)
    g2 = dinv * (_asum_of(a_ref) + ys2_ref[...])
    h2 = jnp.maximum(g2 + b2_ref[...], 0.0)
    ys3_ref[...] = dinv * jnp.dot(h2, w3_ref[...],
                                  preferred_element_type=jnp.float32)


def _layer3_body(dp_ref, ys3_ref, a_ref, b3_ref, out_ref):
    dinv = _dinv_of(dp_ref)
    g3 = dinv * (_asum_of(a_ref) + ys3_ref[...])
    out_ref[...] = g3[:, :40] + b3_ref[...]


def _dp_spec():
    return pl.BlockSpec((2, BLK, 16), lambda i: (0, i, 0))


def _rows_spec(f):
    return pl.BlockSpec((BLK, f), lambda i: (i, 0))


def _agg_spec():
    return pl.BlockSpec((1, 2, BLK, 128), lambda i: (i // 9, 0, i % 9, 0))


def _full_spec(shape):
    nd = len(shape)
    return pl.BlockSpec(shape, lambda i: (0,) * nd)


def _scale(dp, x):
    return pl.pallas_call(
        _scale_body,
        grid=(N // BLK,),
        in_specs=[_dp_spec(), _rows_spec(128)],
        out_specs=_rows_spec(128),
        out_shape=jax.ShapeDtypeStruct((N, 128), jnp.float32),
    )(dp, x)


def _layer1(dp, x, agg1, w1, b1, w2p):
    return pl.pallas_call(
        _layer1_body,
        grid=(N // BLK,),
        in_specs=[_dp_spec(), _rows_spec(128), _agg_spec(),
                  _full_spec((128, 300)), _full_spec((1, 300)),
                  _full_spec((300, 128))],
        out_specs=_rows_spec(128),
        out_shape=jax.ShapeDtypeStruct((N, 128), jnp.float32),
    )(dp, x, agg1, w1, b1, w2p)


def _layer2(dp, ys2, agg2, b2p, w3p):
    return pl.pallas_call(
        _layer2_body,
        grid=(N // BLK,),
        in_specs=[_dp_spec(), _rows_spec(128), _agg_spec(),
                  _full_spec((1, 128)), _full_spec((128, 128))],
        out_specs=_rows_spec(128),
        out_shape=jax.ShapeDtypeStruct((N, 128), jnp.float32),
    )(dp, ys2, agg2, b2p, w3p)


def _layer3(dp, ys3, agg3, b3):
    return pl.pallas_call(
        _layer3_body,
        grid=(N // BLK,),
        in_specs=[_dp_spec(), _rows_spec(128), _agg_spec(),
                  _full_spec((1, 40))],
        out_specs=_rows_spec(40),
        out_shape=jax.ShapeDtypeStruct((N, 40), jnp.float32),
    )(dp, ys3, agg3, b3)


def kernel(x, edge_index, edge_attr, W1, b1, W2, b2, W3, b3):
    src = edge_index[0]
    dst = edge_index[1]
    dst_deg = dst.reshape(NW, NCH, CH)
    srcp = jnp.pad(src.reshape(NW, EPW),
                   ((0, 0), (0, EPWP - EPW))).reshape(NW, NCHP, CHP)
    dstp = jnp.pad(dst.reshape(NW, EPW), ((0, 0), (0, EPWP - EPW)),
                   constant_values=PAD_DST).reshape(NW, NCHP, CHP)

    dp = _deg_call(dst_deg)
    xs = _scale(dp, x)
    agg1 = _spmm(srcp, dstp, xs)

    w2p = jnp.pad(W2, ((0, 0), (0, 78)))
    b2p = jnp.pad(b2, (0, 78)).reshape(1, 128)
    w3p = jnp.pad(W3, ((0, 78), (0, 88)))

    ys2 = _layer1(dp, x, agg1, W1, b1.reshape(1, 300), w2p)
    agg2 = _spmm(srcp, dstp, ys2)
    ys3 = _layer2(dp, ys2, agg2, b2p, w3p)
    agg3 = _spmm(srcp, dstp, ys3)
    return _layer3(dp, ys3, agg3, b3.reshape(1, 40))


# speedup vs baseline: 9.9775x; 1.7931x over previous
import functools

import jax
import jax.numpy as jnp
from jax import lax
from jax.experimental import pallas as pl
from jax.experimental.pallas import tpu as pltpu
from jax.experimental.pallas import tpu_sc as plsc

N = 10000
E = 320000
NC = 2
NS = 16
NW = NC * NS
EPW = E // NW

CH = 125
NCH = EPW // CH
WBD = 80
NWBD = N // WBD

CHP = 64
NR = 80
EPWP = NR * 128

ACC_R = 10240
NZ = ACC_R // CHP
WBR = 200
NWBR = N // WBR


def _mesh():
    return plsc.VectorSubcoreMesh(core_axis_name="c", subcore_axis_name="s")


def _make_degree():
    @functools.partial(
        pl.kernel,
        out_type=jax.ShapeDtypeStruct((NC, N, 16), jnp.float32),
        mesh=_mesh(),
        scratch_types=[
            pltpu.VMEM((NCH, CH), jnp.int32),
            pltpu.VMEM((CH, 16), jnp.float32),
            pltpu.VMEM_SHARED((N, 16), jnp.float32),
        ],
    )
    def deg_kernel(dst_hbm, out_hbm, idxv, buf, acc):
        c = lax.axis_index("c")
        s = lax.axis_index("s")
        wid = s * NC + c

        @pl.loop(0, CH)
        def _zero(i):
            buf[i, :] = jnp.zeros((16,), jnp.float32)

        @pl.loop(s, NWBD, step=NS)
        def _zacc(t):
            pltpu.sync_copy(buf.at[pl.ds(0, WBD)],
                            acc.at[pl.ds(t * WBD, WBD)])

        @pl.loop(0, CH)
        def _fill(i):
            buf[i, :] = jnp.ones((16,), jnp.float32)

        pltpu.sync_copy(dst_hbm.at[wid], idxv)
        plsc.subcore_barrier()

        @pl.loop(0, NCH)
        def _scatter(j):
            pltpu.sync_copy(buf, acc.at[idxv.at[j]], add=True)

        plsc.subcore_barrier()

        @pl.loop(s, NWBD, step=NS)
        def _wb(t):
            pltpu.sync_copy(acc.at[pl.ds(t * WBD, WBD)],
                            out_hbm.at[c, pl.ds(t * WBD, WBD)])

    return deg_kernel


def _make_spmm():
    @functools.partial(
        pl.kernel,
        out_type=jax.ShapeDtypeStruct((NC, N, 128), jnp.float32),
        mesh=_mesh(),
        scratch_types=[
            pltpu.VMEM((NR, 128), jnp.int32),
            pltpu.VMEM((NR, 128), jnp.int32),
            pltpu.VMEM((2, CHP, 128), jnp.float32),
            pltpu.VMEM_SHARED((ACC_R, 128), jnp.float32),
            pltpu.SemaphoreType.DMA,
            pltpu.SemaphoreType.DMA,
        ],
    )
    def spmm(src_hbm, dst_hbm, y_hbm, out_hbm, srcv, dstv,
             rows, acc, sem0, sem1):
        c = lax.axis_index("c")
        s = lax.axis_index("s")
        wid = s * NC + c
        sems = (sem0, sem1)

        pltpu.sync_copy(src_hbm.at[wid], srcv)
        pltpu.sync_copy(dst_hbm.at[wid], dstv)

        @pl.loop(0, CHP)
        def _zero(i):
            for k in range(8):
                rows[0, i, pl.ds(16 * k, 16)] = jnp.zeros((16,), jnp.float32)

        @pl.loop(s, NZ, step=NS)
        def _zacc(t):
            pltpu.sync_copy(rows.at[0], acc.at[pl.ds(t * CHP, CHP)])

        def start_gather(r, h, b):
            pltpu.async_copy(y_hbm.at[srcv.at[r, pl.ds(CHP * h, CHP)]],
                             rows.at[b], sems[b])

        def wait_gather(r, h, b):
            pltpu.make_async_copy(y_hbm.at[srcv.at[r, pl.ds(CHP * h, CHP)]],
                                  rows.at[b], sems[b]).wait()

        plsc.subcore_barrier()

        start_gather(0, 0, 0)

        @pl.loop(0, NR)
        def _edges(r):
            start_gather(r, 1, 1)
            wait_gather(r, 0, 0)
            pltpu.sync_copy(rows.at[0], acc.at[dstv.at[r, pl.ds(0, CHP)]],
                            add=True)

            @pl.when(r + 1 < NR)
            def _():
                start_gather(r + 1, 0, 0)

            wait_gather(r, 1, 1)
            pltpu.sync_copy(rows.at[1], acc.at[dstv.at[r, pl.ds(CHP, CHP)]],
                            add=True)

        plsc.subcore_barrier()

        @pl.loop(s, NWBR, step=NS)
        def _wb(t):
            pltpu.sync_copy(acc.at[pl.ds(t * WBR, WBR)],
                            out_hbm.at[c, pl.ds(t * WBR, WBR)])

    return spmm


_deg_call = _make_degree()
_spmm = _make_spmm()


BLK = 1000


def _dinv_of(dp_ref):
    deg = dp_ref[0, :, 0:1] + dp_ref[1, :, 0:1] + 1.0
    return lax.rsqrt(deg)


def _asum_of(a_ref):
    return a_ref[0] + a_ref[1]


def _scale_body(dp_ref, x_ref, xs_ref):
    xs_ref[...] = x_ref[...] * _dinv_of(dp_ref)


def _layer1_body(dp_ref, x_ref, a_ref, w1_ref, b1_ref, w2_ref, ys2_ref):
    dinv = _dinv_of(dp_ref)
    g1 = dinv * (_asum_of(a_ref) + dinv * x_ref[...])
    h1 = jnp.dot(g1, w1_ref[...], preferred_element_type=jnp.float32)
    h1 = jnp.maximum(h1 + b1_ref[...], 0.0)
    ys2_ref[...] = dinv * jnp.dot(h1, w2_ref[...],
                                  preferred_element_type=jnp.float32)


def _layer2_body(dp_ref, ys2_ref, a_ref, b2_ref, w3_ref, ys3_ref):
    dinv = _dinv_of(dp_ref)
    g2 = dinv * (_asum_of(a_ref) + ys2_ref[...])
    h2 = jnp.maximum(g2 + b2_ref[...], 0.0)
    ys3_ref[...] = dinv * jnp.dot(h2, w3_ref[...],
                                  preferred_element_type=jnp.float32)


def _layer3_body(dp_ref, ys3_ref, a_ref, b3_ref, out_ref):
    dinv = _dinv_of(dp_ref)
    g3 = dinv * (_asum_of(a_ref) + ys3_ref[...])
    out_ref[...] = g3[:, :40] + b3_ref[...]


def _dp_spec():
    return pl.BlockSpec((2, BLK, 16), lambda i: (0, i, 0))


def _rows_spec(f):
    return pl.BlockSpec((BLK, f), lambda i: (i, 0))


def _agg_spec():
    return pl.BlockSpec((NC, BLK, 128), lambda i: (0, i, 0))


def _full_spec(shape):
    nd = len(shape)
    return pl.BlockSpec(shape, lambda i: (0,) * nd)


def _scale(dp, x):
    return pl.pallas_call(
        _scale_body,
        grid=(N // BLK,),
        in_specs=[_dp_spec(), _rows_spec(128)],
        out_specs=_rows_spec(128),
        out_shape=jax.ShapeDtypeStruct((N, 128), jnp.float32),
    )(dp, x)


def _layer1(dp, x, agg1, w1, b1, w2p):
    return pl.pallas_call(
        _layer1_body,
        grid=(N // BLK,),
        in_specs=[_dp_spec(), _rows_spec(128), _agg_spec(),
                  _full_spec((128, 300)), _full_spec((1, 300)),
                  _full_spec((300, 128))],
        out_specs=_rows_spec(128),
        out_shape=jax.ShapeDtypeStruct((N, 128), jnp.float32),
    )(dp, x, agg1, w1, b1, w2p)


def _layer2(dp, ys2, agg2, b2p, w3p):
    return pl.pallas_call(
        _layer2_body,
        grid=(N // BLK,),
        in_specs=[_dp_spec(), _rows_spec(128), _agg_spec(),
                  _full_spec((1, 128)), _full_spec((128, 128))],
        out_specs=_rows_spec(128),
        out_shape=jax.ShapeDtypeStruct((N, 128), jnp.float32),
    )(dp, ys2, agg2, b2p, w3p)


def _layer3(dp, ys3, agg3, b3):
    return pl.pallas_call(
        _layer3_body,
        grid=(N // BLK,),
        in_specs=[_dp_spec(), _rows_spec(128), _agg_spec(),
                  _full_spec((1, 40))],
        out_specs=_rows_spec(40),
        out_shape=jax.ShapeDtypeStruct((N, 40), jnp.float32),
    )(dp, ys3, agg3, b3)


def kernel(x, edge_index, edge_attr, W1, b1, W2, b2, W3, b3):
    src = edge_index[0]
    dst = edge_index[1]
    dst_deg = dst.reshape(NW, NCH, CH)
    srcp = jnp.pad(src.reshape(NW, EPW),
                   ((0, 0), (0, EPWP - EPW))).reshape(NW, NR, 128)
    pad_vals = jnp.broadcast_to(
        jnp.arange(N, N + EPWP - EPW, dtype=jnp.int32),
        (NW, EPWP - EPW))
    dstp = jnp.concatenate([dst.reshape(NW, EPW), pad_vals],
                           axis=1).reshape(NW, NR, 128)

    dp = _deg_call(dst_deg)
    xs = _scale(dp, x)
    agg1 = _spmm(srcp, dstp, xs)

    w2p = jnp.pad(W2, ((0, 0), (0, 78)))
    b2p = jnp.pad(b2, (0, 78)).reshape(1, 128)
    w3p = jnp.pad(W3, ((0, 78), (0, 88)))

    ys2 = _layer1(dp, x, agg1, W1, b1.reshape(1, 300), w2p)
    agg2 = _spmm(srcp, dstp, ys2)
    ys3 = _layer2(dp, ys2, agg2, b2p, w3p)
    agg3 = _spmm(srcp, dstp, ys3)
    return _layer3(dp, ys3, agg3, b3.reshape(1, 40))


# speedup vs baseline: 9.9913x; 1.0014x over previous
import functools

import jax
import jax.numpy as jnp
from jax import lax
from jax.experimental import pallas as pl
from jax.experimental.pallas import tpu as pltpu
from jax.experimental.pallas import tpu_sc as plsc

N = 10000
E = 320000
NC = 2
NS = 16
NW = NC * NS
EPW = E // NW

CH = 125
NCH = EPW // CH
WBD = 80
NWBD = N // WBD

CHP = 64
NR = 80
EPWP = NR * 128

ACC_R = 10240
NZ = ACC_R // CHP
LDR = 200
NLD = N // LDR


def _mesh():
    return plsc.VectorSubcoreMesh(core_axis_name="c", subcore_axis_name="s")


def _make_degree():
    @functools.partial(
        pl.kernel,
        out_type=jax.ShapeDtypeStruct((NC, N, 16), jnp.float32),
        mesh=_mesh(),
        scratch_types=[
            pltpu.VMEM((NCH, CH), jnp.int32),
            pltpu.VMEM((CH, 16), jnp.float32),
            pltpu.VMEM_SHARED((N, 16), jnp.float32),
        ],
    )
    def deg_kernel(dst_hbm, out_hbm, idxv, buf, acc):
        c = lax.axis_index("c")
        s = lax.axis_index("s")
        wid = s * NC + c

        @pl.loop(0, CH)
        def _zero(i):
            buf[i, :] = jnp.zeros((16,), jnp.float32)

        @pl.loop(s, NWBD, step=NS)
        def _zacc(t):
            pltpu.sync_copy(buf.at[pl.ds(0, WBD)],
                            acc.at[pl.ds(t * WBD, WBD)])

        @pl.loop(0, CH)
        def _fill(i):
            buf[i, :] = jnp.ones((16,), jnp.float32)

        pltpu.sync_copy(dst_hbm.at[wid], idxv)
        plsc.subcore_barrier()

        @pl.loop(0, NCH)
        def _scatter(j):
            pltpu.sync_copy(buf, acc.at[idxv.at[j]], add=True)

        plsc.subcore_barrier()

        @pl.loop(s, NWBD, step=NS)
        def _wb(t):
            pltpu.sync_copy(acc.at[pl.ds(t * WBD, WBD)],
                            out_hbm.at[c, pl.ds(t * WBD, WBD)])

    return deg_kernel


def _make_spmm(f):
    @functools.partial(
        pl.kernel,
        out_type=jax.ShapeDtypeStruct((NC, N, f), jnp.float32),
        mesh=_mesh(),
        scratch_types=[
            pltpu.VMEM((NR, 128), jnp.int32),
            pltpu.VMEM((NR, 128), jnp.int32),
            pltpu.VMEM((2, CHP, f), jnp.float32),
            pltpu.VMEM_SHARED((ACC_R, f), jnp.float32),
            pltpu.SemaphoreType.DMA,
            pltpu.SemaphoreType.DMA,
        ],
    )
    def spmm(src_hbm, dst_hbm, y_hbm, out_hbm, srcv, dstv,
             rows, acc, sem0, sem1):
        c = lax.axis_index("c")
        s = lax.axis_index("s")
        wid = s * NC + c
        sems = (sem0, sem1)

        pltpu.sync_copy(src_hbm.at[wid], srcv)
        pltpu.sync_copy(dst_hbm.at[wid], dstv)

        def start_gather(r, q, b):
            pltpu.async_copy(
                y_hbm.at[srcv.at[r, pl.ds(CHP * q, CHP)]],
                rows.at[b], sems[b])

        def wait_gather(r, q, b):
            pltpu.make_async_copy(
                y_hbm.at[srcv.at[r, pl.ds(CHP * q, CHP)]],
                rows.at[b], sems[b]).wait()

        @pl.loop(0, CHP)
        def _zrow(i):
            for k in range(f // 16):
                rows[0, i, pl.ds(16 * k, 16)] = jnp.zeros((16,), jnp.float32)

        @pl.loop(s, NZ, step=NS)
        def _zacc(t):
            pltpu.sync_copy(rows.at[0], acc.at[pl.ds(t * CHP, CHP)])

        plsc.subcore_barrier()

        start_gather(0, 0, 0)

        @pl.loop(0, NR)
        def _edges(r):
            start_gather(r, 1, 1)
            wait_gather(r, 0, 0)
            pltpu.sync_copy(rows.at[0],
                            acc.at[dstv.at[r, pl.ds(0, CHP)]],
                            add=True)

            @pl.when(r + 1 < NR)
            def _():
                start_gather(r + 1, 0, 0)

            wait_gather(r, 1, 1)
            pltpu.sync_copy(rows.at[1],
                            acc.at[dstv.at[r, pl.ds(CHP, CHP)]],
                            add=True)

        plsc.subcore_barrier()

        @pl.loop(s, NLD, step=NS)
        def _wb(t):
            pltpu.sync_copy(acc.at[pl.ds(t * LDR, LDR)],
                            out_hbm.at[c, pl.ds(t * LDR, LDR)])

    return spmm


_deg_call = _make_degree()
_spmm128 = _make_spmm(128)


BLK = 1000


def _dinv_of(dp_ref):
    deg = dp_ref[0, :, 0:1] + dp_ref[1, :, 0:1] + 1.0
    return lax.rsqrt(deg)


def _asum_of(a_ref):
    return a_ref[0] + a_ref[1]


def _scale_body(dp_ref, x_ref, xs_ref):
    xs_ref[...] = x_ref[...] * _dinv_of(dp_ref)


def _layer1_body(dp_ref, x_ref, a_ref, w1_ref, b1_ref, w2_ref, ys2_ref):
    dinv = _dinv_of(dp_ref)
    g1 = dinv * (_asum_of(a_ref) + dinv * x_ref[...])
    h1 = jnp.dot(g1, w1_ref[...], preferred_element_type=jnp.float32)
    h1 = jnp.maximum(h1 + b1_ref[...], 0.0)
    ys2_ref[...] = dinv * jnp.dot(h1, w2_ref[...],
                                  preferred_element_type=jnp.float32)


def _layer2_body(dp_ref, ys2_ref, a_ref, b2_ref, w3_ref, ys3_ref):
    dinv = _dinv_of(dp_ref)
    g2 = dinv * (_asum_of(a_ref) + ys2_ref[...])
    h2 = jnp.maximum(g2 + b2_ref[...], 0.0)
    ys3_ref[...] = dinv * jnp.dot(h2, w3_ref[...],
                                  preferred_element_type=jnp.float32)


def _layer3_body(dp_ref, ys3_ref, a_ref, b3_ref, out_ref):
    dinv = _dinv_of(dp_ref)
    g3 = dinv * (_asum_of(a_ref) + ys3_ref[...])
    out_ref[...] = g3[:, :40] + b3_ref[...]


def _dp_spec():
    return pl.BlockSpec((2, BLK, 16), lambda i: (0, i, 0))


def _rows_spec(f):
    return pl.BlockSpec((BLK, f), lambda i: (i, 0))


def _agg_spec(f):
    return pl.BlockSpec((NC, BLK, f), lambda i: (0, i, 0))


def _full_spec(shape):
    nd = len(shape)
    return pl.BlockSpec(shape, lambda i: (0,) * nd)


def _scale(dp, x):
    return pl.pallas_call(
        _scale_body,
        grid=(N // BLK,),
        in_specs=[_dp_spec(), _rows_spec(128)],
        out_specs=_rows_spec(128),
        out_shape=jax.ShapeDtypeStruct((N, 128), jnp.float32),
    )(dp, x)


def _layer1(dp, x, agg1, w1, b1, w2p):
    return pl.pallas_call(
        _layer1_body,
        grid=(N // BLK,),
        in_specs=[_dp_spec(), _rows_spec(128), _agg_spec(128),
                  _full_spec((128, 300)), _full_spec((1, 300)),
                  _full_spec((300, 128))],
        out_specs=_rows_spec(128),
        out_shape=jax.ShapeDtypeStruct((N, 128), jnp.float32),
    )(dp, x, agg1, w1, b1, w2p)


def _layer2(dp, ys2, agg2, b2p, w3p):
    return pl.pallas_call(
        _layer2_body,
        grid=(N // BLK,),
        in_specs=[_dp_spec(), _rows_spec(128), _agg_spec(128),
                  _full_spec((1, 128)), _full_spec((128, 128))],
        out_specs=_rows_spec(128),
        out_shape=jax.ShapeDtypeStruct((N, 128), jnp.float32),
    )(dp, ys2, agg2, b2p, w3p)


def _layer3(dp, ys3, agg3, b3):
    return pl.pallas_call(
        _layer3_body,
        grid=(N // BLK,),
        in_specs=[_dp_spec(), _rows_spec(128), _agg_spec(128),
                  _full_spec((1, 40))],
        out_specs=_rows_spec(40),
        out_shape=jax.ShapeDtypeStruct((N, 40), jnp.float32),
    )(dp, ys3, agg3, b3)


def kernel(x, edge_index, edge_attr, W1, b1, W2, b2, W3, b3):
    src = edge_index[0]
    dst = edge_index[1]
    dst_deg = dst.reshape(NW, NCH, CH)
    srcp = jnp.pad(src.reshape(NW, EPW),
                   ((0, 0), (0, EPWP - EPW))).reshape(NW, NR, 128)
    pad_vals = jnp.broadcast_to(
        jnp.arange(N, N + EPWP - EPW, dtype=jnp.int32),
        (NW, EPWP - EPW))
    dstp = jnp.concatenate([dst.reshape(NW, EPW), pad_vals],
                           axis=1).reshape(NW, NR, 128)

    dp = _deg_call(dst_deg)
    xs = _scale(dp, x)
    agg1 = _spmm128(srcp, dstp, xs)

    w2p = jnp.pad(W2, ((0, 0), (0, 78)))
    b2p = jnp.pad(b2, (0, 78)).reshape(1, 128)
    w3p = jnp.pad(W3, ((0, 78), (0, 88)))

    ys2 = _layer1(dp, x, agg1, W1, b1.reshape(1, 300), w2p)
    agg2 = _spmm128(srcp, dstp, ys2)
    ys3 = _layer2(dp, ys2, agg2, b2p, w3p)
    agg3 = _spmm128(srcp, dstp, ys3)
    return _layer3(dp, ys3, agg3, b3.reshape(1, 40))
